# baseline (speedup 1.0000x reference)
import functools

import jax
import jax.numpy as jnp
from jax import lax
from jax.experimental import pallas as pl
from jax.experimental.pallas import tpu as pltpu
from jax.experimental.pallas import tpu_sc as plsc

N_NODES = 10000
N_PAIRS = N_NODES // 2
N_EDGES = 160000
D = 256
DQ = D // 4
EPS = 1e-5

LANES = 16
E_BLK = 80
N_CHUNKS = N_EDGES // E_BLK
NSC = 16
MAX_STEPS = N_CHUNKS // NSC
NB = 2

_HI_MASK = -65536

_GDN = lax.GatherDimensionNumbers(
    offset_dims=(), collapsed_slice_dims=(0,), start_index_map=(0,))


def _bcast(vec, idx):
    return lax.gather(vec, idx[:, None], _GDN, (1,),
                      mode=lax.GatherScatterMode.PROMISE_IN_BOUNDS)


def _ln_body(x_ref, g_ref, b_ref, o_ref):
    x = x_ref[...]
    mean = jnp.mean(x, axis=-1, keepdims=True)
    var = jnp.mean((x - mean) ** 2, axis=-1, keepdims=True)
    xn = (x - mean) * lax.rsqrt(var + EPS) * g_ref[...] + b_ref[...]

    def pack(lo, hi):
        lo16 = lax.bitcast_convert_type(lo.astype(jnp.bfloat16), jnp.uint16)
        hi16 = lax.bitcast_convert_type(hi.astype(jnp.bfloat16), jnp.uint16)
        w = lo16.astype(jnp.uint32) | (hi16.astype(jnp.uint32) << 16)
        return lax.bitcast_convert_type(w, jnp.int32)

    for c in range(2):
        cs = c * 128
        pa = pack(xn[0, :, cs:cs + 64], xn[0, :, cs + 64:cs + 128])
        pb = pack(xn[1, :, cs:cs + 64], xn[1, :, cs + 64:cs + 128])
        o_ref[c] = jnp.concatenate([pa, pb], axis=1)


def _layernorm_packed(x, gamma, beta):
    blk = 1000
    x2 = x.reshape(2, N_PAIRS, D)
    return pl.pallas_call(
        _ln_body,
        grid=(N_PAIRS // blk,),
        in_specs=[
            pl.BlockSpec((2, blk, D), lambda i: (0, i, 0)),
            pl.BlockSpec((D,), lambda i: (0,)),
            pl.BlockSpec((D,), lambda i: (0,)),
        ],
        out_specs=pl.BlockSpec((2, blk, 2 * DQ), lambda i: (0, i, 0)),
        out_shape=jax.ShapeDtypeStruct((2, N_PAIRS, 2 * DQ), jnp.int32),
    )(x2, gamma, beta)


_MESH = plsc.VectorSubcoreMesh(core_axis_name="c", subcore_axis_name="s")


@functools.partial(
    pl.kernel,
    out_type=jax.ShapeDtypeStruct((N_EDGES, D), jnp.float32),
    mesh=_MESH,
    scratch_types=(
        [pltpu.VMEM((E_BLK,), jnp.int32)] * (4 * NB)
        + [pltpu.VMEM((E_BLK, 2 * DQ), jnp.int32)] * (2 * NB)
        + [pltpu.VMEM((E_BLK, 2 * DQ), jnp.float32)] * NB
        + [pltpu.VMEM_SHARED((N_PAIRS, 2 * DQ), jnp.int32)]
        + [pltpu.SemaphoreType.DMA] * (3 * NB)
    ),
)
def _gather_mul(xn_hbm, src_hbm, dst_hbm, out_hbm,
                si0, di0, pa0, pb0, si1, di1, pa1, pb1,
                a0, b0, a1, b1, o0, o1, tab,
                gi0, gi1, g0, g1, w0, w1):
    cid = lax.axis_index("c")
    sid = lax.axis_index("s")

    rows = 312

    @pl.when(sid < 15)
    def _():
        pltpu.sync_copy(xn_hbm.at[cid, pl.ds(sid * rows, rows)],
                        tab.at[pl.ds(sid * rows, rows)])

    @pl.when(sid == 15)
    def _():
        pltpu.sync_copy(xn_hbm.at[cid, pl.ds(15 * rows, N_PAIRS - 15 * rows)],
                        tab.at[pl.ds(15 * rows, N_PAIRS - 15 * rows)])

    plsc.subcore_barrier()

    SI, DI = (si0, si1), (di0, di1)
    PA, PB = (pa0, pa1), (pb0, pb1)
    A, B, O = (a0, a1), (b0, b1), (o0, o1)
    GI, G, W = (gi0, gi1), (g0, g1), (w0, w1)
    col0 = cid * (2 * DQ)

    def cbase(s):
        return (sid + s * NSC) * E_BLK

    def valid(s):
        return (sid + s * NSC) < N_CHUNKS

    def fire_idx(s, k):
        pltpu.async_copy(src_hbm.at[pl.ds(cbase(s), E_BLK)], SI[k], GI[k])
        pltpu.async_copy(dst_hbm.at[pl.ds(cbase(s), E_BLK)], DI[k], GI[k])

    def wait_idx(k):
        pltpu.make_async_copy(src_hbm.at[pl.ds(0, E_BLK)], SI[k], GI[k]).wait()
        pltpu.make_async_copy(dst_hbm.at[pl.ds(0, E_BLK)], DI[k], GI[k]).wait()

    def split_idx(k):
        sik, dik, pak, pbk = SI[k], DI[k], PA[k], PB[k]

        def _vec(v, _):
            sl = pl.ds(v * LANES, LANES)
            s_raw = sik[sl]
            d_raw = dik[sl]
            s_hi = ((s_raw - N_PAIRS) >> 31) + 1
            d_hi = ((d_raw - N_PAIRS) >> 31) + 1
            sik[sl] = s_raw - s_hi * N_PAIRS
            dik[sl] = d_raw - d_hi * N_PAIRS
            pak[sl] = s_hi
            pbk[sl] = d_hi
            return 0

        lax.fori_loop(0, E_BLK // LANES, _vec, 0)

    def fire_gathers(k):
        pltpu.async_copy(tab.at[SI[k]], A[k], G[k])
        pltpu.async_copy(tab.at[DI[k]], B[k], G[k])

    def wait_gathers(k):
        pltpu.make_async_copy(tab.at[pl.ds(0, E_BLK)], A[k], G[k]).wait()
        pltpu.make_async_copy(tab.at[pl.ds(0, E_BLK)], B[k], G[k]).wait()

    def fire_wb(s, k):
        pltpu.async_copy(O[k], out_hbm.at[pl.ds(cbase(s), E_BLK),
                                          pl.ds(col0, 2 * DQ)], W[k])

    def wait_wb(k):
        pltpu.make_async_copy(O[k], out_hbm.at[pl.ds(0, E_BLK),
                                               pl.ds(col0, 2 * DQ)], W[k]).wait()

    def multiply(k):
        ak, bk, ok, pak, pbk = A[k], B[k], O[k], PA[k], PB[k]

        @plsc.parallel_loop(0, E_BLK, unroll=4)
        def _row(e):
            base = (e // LANES) * LANES
            lanev = jnp.zeros((LANES,), jnp.int32) + (e - base)
            pa16 = pak[pl.ds(base, LANES)]
            pb16 = pbk[pl.ds(base, LANES)]
            ma = _bcast(pa16, lanev)
            mb = _bcast(pb16, lanev)
            for j in range(DQ // LANES):
                sl = pl.ds(j * LANES, LANES)
                sh = pl.ds(DQ + j * LANES, LANES)
                wal = ak[e, sl]
                wbl = bk[e, sl]
                wa = wal + (ak[e, sh] - wal) * ma
                wb = wbl + (bk[e, sh] - wbl) * mb
                a_lo = lax.bitcast_convert_type(wa << 16, jnp.float32)
                b_lo = lax.bitcast_convert_type(wb << 16, jnp.float32)
                a_hi = lax.bitcast_convert_type(wa & _HI_MASK, jnp.float32)
                b_hi = lax.bitcast_convert_type(wb & _HI_MASK, jnp.float32)
                ok[e, sl] = a_lo * b_lo
                ok[e, sh] = a_hi * b_hi

    fire_idx(0, 0)
    fire_idx(1, 1)
    wait_idx(0)
    split_idx(0)
    fire_gathers(0)

    def step(s, k):
        kn = 1 - k

        @pl.when(jnp.logical_and(s >= 1, valid(s - 1)))
        def _():
            wait_wb(kn)

        @pl.when(valid(s + 1))
        def _():
            wait_idx(kn)
            split_idx(kn)
            fire_gathers(kn)

        @pl.when(valid(s))
        def _():
            wait_gathers(k)

        @pl.when(valid(s + 2))
        def _():
            fire_idx(s + 2, k)

        @pl.when(valid(s))
        def _():
            multiply(k)
            fire_wb(s, k)

    def pair(i, _):
        step(2 * i, 0)
        step(2 * i + 1, 1)
        return 0

    lax.fori_loop(0, (MAX_STEPS + 2) // 2, pair, 0)


def kernel(data, x, edge, gamma, beta):
    xn_packed = _layernorm_packed(x, gamma, beta)
    src = edge[0]
    dst = edge[1]
    return _gather_mul(xn_packed, src, dst)


# speedup vs baseline: 1.3119x; 1.3119x over previous
import functools

import jax
import jax.numpy as jnp
from jax import lax
from jax.experimental import pallas as pl
from jax.experimental.pallas import tpu as pltpu
from jax.experimental.pallas import tpu_sc as plsc

N_NODES = 10000
N_EDGES = 160000
D = 256
DH = D // 2
EPS = 1e-5

LANES = 16
E_BLK = 80
N_CHUNKS = N_EDGES // E_BLK
NW = 32
MAX_STEPS = -(-N_CHUNKS // NW)
NB = 3

_HI_MASK = -65536


def _ln_body(x_ref, g_ref, b_ref, o_ref):
    x = x_ref[...]
    mean = jnp.mean(x, axis=-1, keepdims=True)
    var = jnp.mean((x - mean) ** 2, axis=-1, keepdims=True)
    xn = (x - mean) * lax.rsqrt(var + EPS) * g_ref[...] + b_ref[...]
    lo = lax.bitcast_convert_type(xn[:, :DH].astype(jnp.bfloat16), jnp.uint16)
    hi = lax.bitcast_convert_type(xn[:, DH:].astype(jnp.bfloat16), jnp.uint16)
    packed = lo.astype(jnp.uint32) | (hi.astype(jnp.uint32) << 16)
    o_ref[...] = lax.bitcast_convert_type(packed, jnp.int32)


def _layernorm_packed(x, gamma, beta):
    blk = 5000
    return pl.pallas_call(
        _ln_body,
        grid=(N_NODES // blk,),
        in_specs=[
            pl.BlockSpec((blk, D), lambda i: (i, 0)),
            pl.BlockSpec((D,), lambda i: (0,)),
            pl.BlockSpec((D,), lambda i: (0,)),
        ],
        out_specs=pl.BlockSpec((blk, DH), lambda i: (i, 0)),
        out_shape=jax.ShapeDtypeStruct((N_NODES, DH), jnp.int32),
    )(x, gamma, beta)


_MESH = plsc.VectorSubcoreMesh(core_axis_name="c", subcore_axis_name="s")


@functools.partial(
    pl.kernel,
    out_type=jax.ShapeDtypeStruct((N_EDGES, D), jnp.float32),
    mesh=_MESH,
    scratch_types=(
        [pltpu.VMEM((E_BLK,), jnp.int32)] * (2 * NB)
        + [pltpu.VMEM((E_BLK, DH), jnp.int32)] * (2 * NB)
        + [pltpu.VMEM((E_BLK, D), jnp.float32)] * NB
        + [pltpu.SemaphoreType.DMA] * (3 * NB)
    ),
)
def _gather_mul(xn_hbm, src_hbm, dst_hbm, out_hbm,
                si0, di0, si1, di1, si2, di2,
                a0, b0, a1, b1, a2, b2, o0, o1, o2,
                gi0, gi1, gi2, g0, g1, g2, w0, w1, w2):
    wid = lax.axis_index("s") * 2 + lax.axis_index("c")
    SI, DI = (si0, si1, si2), (di0, di1, di2)
    A, B, O = (a0, a1, a2), (b0, b1, b2), (o0, o1, o2)
    GI, G, W = (gi0, gi1, gi2), (g0, g1, g2), (w0, w1, w2)

    def cbase(s):
        return (wid + s * NW) * E_BLK

    def valid(s):
        return (wid + s * NW) < N_CHUNKS

    def fire_idx(s, k):
        pltpu.async_copy(src_hbm.at[pl.ds(cbase(s), E_BLK)], SI[k], GI[k])
        pltpu.async_copy(dst_hbm.at[pl.ds(cbase(s), E_BLK)], DI[k], GI[k])

    def wait_idx(k):
        pltpu.make_async_copy(src_hbm.at[pl.ds(0, E_BLK)], SI[k], GI[k]).wait()
        pltpu.make_async_copy(dst_hbm.at[pl.ds(0, E_BLK)], DI[k], GI[k]).wait()

    def fire_gathers(k):
        pltpu.async_copy(xn_hbm.at[SI[k]], A[k], G[k])
        pltpu.async_copy(xn_hbm.at[DI[k]], B[k], G[k])

    def wait_gathers(k):
        pltpu.make_async_copy(xn_hbm.at[pl.ds(0, E_BLK)], A[k], G[k]).wait()
        pltpu.make_async_copy(xn_hbm.at[pl.ds(0, E_BLK)], B[k], G[k]).wait()

    def fire_wb(s, k):
        pltpu.async_copy(O[k], out_hbm.at[pl.ds(cbase(s), E_BLK)], W[k])

    def wait_wb(k):
        pltpu.make_async_copy(O[k], out_hbm.at[pl.ds(0, E_BLK)], W[k]).wait()

    def multiply(k):
        ak, bk, ok = A[k], B[k], O[k]

        @plsc.parallel_loop(0, E_BLK, unroll=4)
        def _row(e):
            for j in range(DH // LANES):
                sl = pl.ds(j * LANES, LANES)
                wa = ak[e, sl]
                wb = bk[e, sl]
                a_lo = lax.bitcast_convert_type(wa << 16, jnp.float32)
                b_lo = lax.bitcast_convert_type(wb << 16, jnp.float32)
                a_hi = lax.bitcast_convert_type(wa & _HI_MASK, jnp.float32)
                b_hi = lax.bitcast_convert_type(wb & _HI_MASK, jnp.float32)
                ok[e, sl] = a_lo * b_lo
                ok[e, pl.ds(DH + j * LANES, LANES)] = a_hi * b_hi

    fire_idx(0, 0)
    fire_idx(1, 1)
    fire_idx(2, 2)
    wait_idx(0)
    fire_gathers(0)
    wait_idx(1)
    fire_gathers(1)

    def step(s, k):
        k2 = (k + 2) % NB

        @pl.when(valid(s + 2))
        def _():
            wait_idx(k2)
            fire_gathers(k2)

        @pl.when(valid(s))
        def _():
            wait_gathers(k)

        @pl.when(valid(s + 3))
        def _():
            fire_idx(s + 3, k)

        @pl.when(jnp.logical_and(s >= 3, valid(s - 3)))
        def _():
            wait_wb(k)

        @pl.when(valid(s))
        def _():
            multiply(k)
            fire_wb(s, k)

    def triple(i, _):
        s0 = 3 * i
        step(s0, 0)
        step(s0 + 1, 1)
        step(s0 + 2, 2)
        return 0

    lax.fori_loop(0, (MAX_STEPS + 3 + 2) // 3, triple, 0)


def kernel(data, x, edge, gamma, beta):
    xn_packed = _layernorm_packed(x, gamma, beta)
    src = edge[0]
    dst = edge[1]
    return _gather_mul(xn_packed, src, dst)
